# baseline (speedup 1.0000x reference)
import functools

import jax
import jax.numpy as jnp
from jax import lax
from jax.experimental import pallas as pl
from jax.experimental.pallas import tpu as pltpu
from jax.experimental.pallas import tpu_sc as plsc

N_USERS = 1000000
N_ITEMS = 1000000
DIM = 32
BATCH = 16384

_INFO = plsc.get_sparse_core_info()
_NC, _NS, _L = _INFO.num_cores, _INFO.num_subcores, _INFO.num_lanes
_NW = _NC * _NS
_BPW = BATCH // _NW
_CHUNK = 128
_NCHUNK = _BPW // _CHUNK

_mesh = plsc.VectorSubcoreMesh(core_axis_name="c", subcore_axis_name="s")


@functools.partial(
    pl.kernel,
    mesh=_mesh,
    out_type=jax.ShapeDtypeStruct((BATCH,), jnp.float32),
    compiler_params=pltpu.CompilerParams(
        needs_layout_passes=False, use_tc_tiling_on_sc=False),
    scratch_types=[
        pltpu.VMEM((_NCHUNK, _CHUNK), jnp.int32),
        pltpu.VMEM((_NCHUNK, _CHUNK), jnp.int32),
        pltpu.VMEM((_NCHUNK, _CHUNK), jnp.int32),
        pltpu.VMEM((_BPW, DIM), jnp.float32),
        pltpu.VMEM((_BPW, DIM), jnp.float32),
        pltpu.VMEM((_BPW, DIM), jnp.float32),
        pltpu.VMEM((_BPW,), jnp.float32),
        pltpu.SemaphoreType.DMA,
    ],
)
def _bpr_sc(ui_hbm, pi_hbm, ni_hbm, gu_hbm, gi_hbm, out_hbm,
            ui_v, pi_v, ni_v, u_rows, p_rows, n_rows, out_v, sem):
    wid = lax.axis_index("s") * _NC + lax.axis_index("c")
    base = wid * _BPW

    pltpu.sync_copy(ui_hbm.at[wid], ui_v)
    pltpu.sync_copy(pi_hbm.at[wid], pi_v)
    pltpu.sync_copy(ni_hbm.at[wid], ni_v)

    copies = []
    for j in range(_NCHUNK):
        dst = pl.ds(j * _CHUNK, _CHUNK)
        copies.append(pltpu.async_copy(gu_hbm.at[ui_v.at[j]], u_rows.at[dst], sem))
        copies.append(pltpu.async_copy(gi_hbm.at[pi_v.at[j]], p_rows.at[dst], sem))
        copies.append(pltpu.async_copy(gi_hbm.at[ni_v.at[j]], n_rows.at[dst], sem))
    for c in copies:
        c.wait()

    lanes = lax.iota(jnp.int32, _L)

    def group(g, _):
        r0 = pl.multiple_of(g * _L, _L)
        acc = jnp.zeros((_L,), jnp.float32)
        for k in range(_L):
            r = r0 + k
            lo, hi = pl.ds(0, _L), pl.ds(_L, _L)
            t = (u_rows[r, lo] * (p_rows[r, lo] - n_rows[r, lo])
                 + u_rows[r, hi] * (p_rows[r, hi] - n_rows[r, hi]))
            acc = jnp.where(lanes == k, jnp.sum(t), acc)
        out_v[pl.ds(r0, _L)] = acc
        return 0

    lax.fori_loop(0, _BPW // _L, group, 0)

    pltpu.sync_copy(out_v, out_hbm.at[pl.ds(base, _BPW)])


def kernel(ui, pi, ni, gamma_users, gamma_items):
    ui3 = ui.astype(jnp.int32).reshape(_NW, _NCHUNK, _CHUNK)
    pi3 = pi.astype(jnp.int32).reshape(_NW, _NCHUNK, _CHUNK)
    ni3 = ni.astype(jnp.int32).reshape(_NW, _NCHUNK, _CHUNK)
    return _bpr_sc(ui3, pi3, ni3, gamma_users, gamma_items)
